# baseline (speedup 1.0000x reference)
import functools

import jax
import jax.numpy as jnp
from jax import lax
from jax.experimental import pallas as pl
from jax.experimental.pallas import tpu as pltpu
from jax.experimental.pallas import tpu_sc as plsc

BATCH = 16384
DIM = 32
VOCAB = 1000000
PACK = 1
PROWS = VOCAB // PACK
NUM_CORES = 2
NUM_SUBCORES = 16
NUM_WORKERS = NUM_CORES * NUM_SUBCORES
BPW = BATCH // NUM_WORKERS
B_CH = 128
NCH = BPW // B_CH
REPACK_BLK = 2048


def _sc_bpr(user, item_i, item_j, pu, pi):
    mesh = plsc.VectorSubcoreMesh(core_axis_name="c", subcore_axis_name="s")

    @functools.partial(
        pl.kernel,
        mesh=mesh,
        out_type=jax.ShapeDtypeStruct((BATCH,), jnp.float32),
        scratch_types=[
            pltpu.VMEM((BPW,), jnp.int32),
            pltpu.VMEM((BPW,), jnp.int32),
            pltpu.VMEM((BPW,), jnp.int32),
            pltpu.VMEM((BPW,), jnp.float32),
            pltpu.VMEM((2, B_CH), jnp.int32),
            pltpu.VMEM((2, B_CH), jnp.int32),
            pltpu.VMEM((2, B_CH), jnp.int32),
            pltpu.VMEM((2, B_CH, DIM), jnp.float32),
            pltpu.VMEM((2, B_CH, DIM), jnp.float32),
            pltpu.VMEM((2, B_CH, DIM), jnp.float32),
            pltpu.SemaphoreType.DMA,
            pltpu.SemaphoreType.DMA,
            pltpu.SemaphoreType.DMA,
        ],
        compiler_params=pltpu.CompilerParams(
            use_tc_tiling_on_sc=False, needs_layout_passes=False
        ),
    )
    def k(u_hbm, i_hbm, j_hbm, pu_hbm, pi_hbm, out_hbm,
          uidx, iidx, jidx, d_v, gqu, gqi, gqj, Gu, Gi, Gj,
          isem, sem0, sem1):
        wid = lax.axis_index("s") * NUM_CORES + lax.axis_index("c")
        base = wid * BPW
        sl = pl.ds(base, BPW)
        cu = pltpu.async_copy(u_hbm.at[sl], uidx, isem)
        ci = pltpu.async_copy(i_hbm.at[sl], iidx, isem)
        cj = pltpu.async_copy(j_hbm.at[sl], jidx, isem)
        cu.wait()
        ci.wait()
        cj.wait()

        iota16 = lax.iota(jnp.int32, 16)

        def gen(ch, parity):
            for g in range(B_CH // 16):
                b0 = ch * B_CH + g * 16
                for idx_ref, gq in ((uidx, gqu), (iidx, gqi), (jidx, gqj)):
                    r16 = idx_ref[pl.ds(b0, 16)]
                    gq.at[parity][pl.ds(g * 16, 16)] = (
                        lax.shift_right_logical(r16, 0)
                    )

        def start(parity, sem):
            pltpu.async_copy(pu_hbm.at[gqu.at[parity]], Gu.at[parity], sem)
            pltpu.async_copy(pi_hbm.at[gqi.at[parity]], Gi.at[parity], sem)
            pltpu.async_copy(pi_hbm.at[gqj.at[parity]], Gj.at[parity], sem)

        def wait(parity, sem):
            pltpu.make_async_copy(
                pu_hbm.at[gqu.at[parity]], Gu.at[parity], sem).wait()
            pltpu.make_async_copy(
                pi_hbm.at[gqi.at[parity]], Gi.at[parity], sem).wait()
            pltpu.make_async_copy(
                pi_hbm.at[gqj.at[parity]], Gj.at[parity], sem).wait()

        def extract(ch, parity):
            for g in range(B_CH // 16):
                b0 = ch * B_CH + g * 16
                rows = g * 16 + iota16
                ru = uidx[pl.ds(b0, 16)]
                ri = iidx[pl.ds(b0, 16)]
                rj = jidx[pl.ds(b0, 16)]
                lu = lax.shift_left(lax.bitwise_and(ru, PACK - 1), 5)
                li = lax.shift_left(lax.bitwise_and(ri, PACK - 1), 5)
                lj = lax.shift_left(lax.bitwise_and(rj, PACK - 1), 5)
                acc = jnp.zeros((16,), jnp.float32)
                for c in range(DIM):
                    vu = plsc.load_gather(Gu.at[parity], [rows, lu + c])
                    vi = plsc.load_gather(Gi.at[parity], [rows, li + c])
                    vj = plsc.load_gather(Gj.at[parity], [rows, lj + c])
                    acc = acc + vu * (vi - vj)
                d_v[pl.ds(b0, 16)] = acc

        gen(0, 0)
        start(0, sem0)

        @pl.loop(0, NCH // 2)
        def _(kk):
            c0 = kk * 2
            c1 = c0 + 1
            gen(c1, 1)
            start(1, sem1)
            wait(0, sem0)
            extract(c0, 0)

            @pl.when(kk < NCH // 2 - 1)
            def _():
                gen(c0 + 2, 0)
                start(0, sem0)

            wait(1, sem1)
            extract(c1, 1)

        pltpu.sync_copy(d_v, out_hbm.at[sl])

    return k(user, item_i, item_j, pu, pi)


def _loss_body(d_ref, o_ref):
    x = -d_ref[...]
    sp = jnp.maximum(x, 0.0) + jnp.log1p(jnp.exp(-jnp.abs(x)))
    o_ref[0, 0] = jnp.sum(sp)


def kernel(user, item_i, item_j, user_emb, item_emb):
    d = _sc_bpr(user, item_i, item_j, user_emb, item_emb)
    loss = pl.pallas_call(
        _loss_body,
        out_shape=jax.ShapeDtypeStruct((1, 1), jnp.float32),
        out_specs=pl.BlockSpec(memory_space=pltpu.SMEM),
    )(d.reshape(128, 128))
    return loss[0, 0]
